# Initial kernel scaffold; baseline (speedup 1.0000x reference)
import jax
import jax.numpy as jnp
from jax.experimental import pallas as pl


def kernel(bbox_pred, conf_pred, anchors, target_boxes):
    raise NotImplementedError("write your pallas kernel here")


# speedup vs baseline: 18.6004x; 18.6004x over previous
import functools

import jax
import jax.numpy as jnp
from jax.experimental import pallas as pl
from jax.experimental.pallas import tpu as pltpu

_N = 100000
_T = 20
_LANES = 128
_R = 784
_NPAD = _R * _LANES


def _loss_kernel(tb_ref, ax_ref, bbox_ref, conf_ref, conf_out_ref, bbox_out_ref,
                 miou_ref, bidx_ref, vm_ref, nbits_ref):
    f32 = jnp.float32
    row = jax.lax.broadcasted_iota(jnp.int32, (_R, _LANES), 0)
    col = jax.lax.broadcasted_iota(jnp.int32, (_R, _LANES), 1)
    lin = row * _LANES + col
    valid = lin < _N

    tb = tb_ref[0]
    ax1 = ax_ref[0]
    ay1 = ax_ref[1]
    ax2 = ax_ref[2]
    ay2 = ax_ref[3]
    a1 = (ax2 - ax1) * (ay2 - ay1)

    tops = []
    for t in range(_T):
        x1t = tb[t, 0]
        y1t = tb[t, 1]
        x2t = tb[t, 2]
        y2t = tb[t, 3]
        a2 = (x2t - x1t) * (y2t - y1t)
        ix1 = jnp.maximum(ax1, x1t)
        iy1 = jnp.maximum(ay1, y1t)
        ix2 = jnp.minimum(ax2, x2t)
        iy2 = jnp.minimum(ay2, y2t)
        inter = jnp.maximum(ix2 - ix1, 0.0) * jnp.maximum(iy2 - iy1, 0.0)
        union = a1 + a2 - inter
        iou_t = inter / (union + 1e-06)
        if t == 0:
            miou_ref[...] = iou_t
            bidx_ref[...] = jnp.zeros((_R, _LANES), jnp.int32)
        else:
            cur = miou_ref[...]
            upd = iou_t > cur
            miou_ref[...] = jnp.where(upd, iou_t, cur)
            bidx_ref[...] = jnp.where(upd, t, bidx_ref[...])
        vm_ref[...] = jnp.where(valid, iou_t, -1.0)
        for k in range(3):
            vm = vm_ref[...]
            m = jnp.max(vm)
            idx = jnp.min(jnp.where(vm == m, lin, _NPAD))
            sel = True if k == 0 else (m > 0.3)
            tops.append((idx, sel))
            vm_ref[...] = jnp.where(lin == idx, -2.0, vm)

    max_iou = miou_ref[...]
    pos = (max_iou >= 0.5) & valid
    for idx, sel in tops:
        hit = lin == idx
        if sel is True:
            pos = pos | hit
        else:
            pos = pos | (hit & sel)
    neg = (max_iou < 0.4) & (~pos) & valid

    conf = conf_ref[0, 0]
    pt = jnp.where(pos, conf, 1.0 - conf)
    af = jnp.where(pos, 0.25, 0.75)
    fl = -af * (1.0 - pt) ** 2 * jnp.log(jnp.clip(pt, 1e-06, 1.0 - 1e-06))

    num_pos = jnp.sum(pos.astype(jnp.int32))
    num_neg = jnp.sum(neg.astype(jnp.int32))
    sum_fl_pos = jnp.sum(jnp.where(pos, fl, 0.0))
    nbits_ref[...] = jnp.where(neg, jax.lax.bitcast_convert_type(fl, jnp.int32),
                               jnp.int32(-1))

    bidx = bidx_ref[...]
    mx1 = jnp.full((_R, _LANES), tb[0, 0])
    my1 = jnp.full((_R, _LANES), tb[0, 1])
    mx2 = jnp.full((_R, _LANES), tb[0, 2])
    my2 = jnp.full((_R, _LANES), tb[0, 3])
    for t in range(1, _T):
        m = bidx == t
        mx1 = jnp.where(m, tb[t, 0], mx1)
        my1 = jnp.where(m, tb[t, 1], my1)
        mx2 = jnp.where(m, tb[t, 2], mx2)
        my2 = jnp.where(m, tb[t, 3], my2)
    px1 = bbox_ref[0, 0]
    py1 = bbox_ref[0, 1]
    px2 = bbox_ref[0, 2]
    py2 = bbox_ref[0, 3]
    ix1 = jnp.maximum(px1, mx1)
    iy1 = jnp.maximum(py1, my1)
    ix2 = jnp.minimum(px2, mx2)
    iy2 = jnp.minimum(py2, my2)
    inter = jnp.maximum(ix2 - ix1, 0.0) * jnp.maximum(iy2 - iy1, 0.0)
    pa = (px2 - px1) * (py2 - py1)
    ma = (mx2 - mx1) * (my2 - my1)
    union = pa + ma - inter
    iou = inter / (union + 1e-06)
    ex1 = jnp.minimum(px1, mx1)
    ey1 = jnp.minimum(py1, my1)
    ex2 = jnp.maximum(px2, mx2)
    ey2 = jnp.maximum(py2, my2)
    enc = (ex2 - ex1) * (ey2 - ey1)
    giou = iou - (enc - union) / (enc + 1e-06)
    l1 = (jnp.abs(px1 - mx1) + jnp.abs(py1 - my1) + jnp.abs(px2 - mx2)
          + jnp.abs(py2 - my2)) * 0.25
    per_anchor = (1.0 - giou) + 0.5 * l1
    sum_bbox_pos = jnp.sum(jnp.where(pos, per_anchor, 0.0))

    safe_np = jnp.maximum(num_pos, 1)
    neg_ratio = jnp.minimum(3, _N // safe_np)
    max_neg = neg_ratio * num_pos
    k1 = jnp.minimum(max_neg, num_neg)
    k2 = jnp.minimum(100, num_neg)

    def bs_body(_, carry):
        lo1, hi1, lo2, hi2 = carry
        m1 = lo1 + (hi1 - lo1) // 2
        m2 = lo2 + (hi2 - lo2) // 2
        nb = nbits_ref[...]
        c1 = jnp.sum((nb >= m1).astype(jnp.int32))
        c2 = jnp.sum((nb >= m2).astype(jnp.int32))
        t1 = c1 >= k1
        t2 = c2 >= k2
        return (jnp.where(t1, m1, lo1), jnp.where(t1, hi1, m1),
                jnp.where(t2, m2, lo2), jnp.where(t2, hi2, m2))

    z = jnp.int32(0)
    top = jnp.int32(0x7F800001)
    lo1, _, lo2, _ = jax.lax.fori_loop(0, 31, bs_body, (z, top, z, top))

    nb = nbits_ref[...]
    nv = jnp.where(nb >= 0, jax.lax.bitcast_convert_type(nb, f32), 0.0)
    v1 = jax.lax.bitcast_convert_type(lo1, f32)
    v2 = jax.lax.bitcast_convert_type(lo2, f32)
    g1 = nb > lo1
    g2 = nb > lo2
    cnt_g1 = jnp.sum(g1.astype(jnp.int32))
    cnt_g2 = jnp.sum(g2.astype(jnp.int32))
    sum_g1 = jnp.sum(jnp.where(g1, nv, 0.0))
    sum_g2 = jnp.sum(jnp.where(g2, nv, 0.0))
    sum_top1 = jnp.where(k1 > 0, sum_g1 + (k1 - cnt_g1).astype(f32) * v1, 0.0)
    sum_top2 = jnp.where(k2 > 0, sum_g2 + (k2 - cnt_g2).astype(f32) * v2, 0.0)

    conf_pos = (sum_fl_pos + sum_top1) / (num_pos + k1).astype(f32)
    conf_neg = sum_top2 / k2.astype(f32)
    conf_i = jnp.where(num_pos > 0, conf_pos, conf_neg)
    bbox_i = jnp.where(num_pos > 0,
                       sum_bbox_pos / safe_np.astype(f32), 0.0)
    conf_out_ref[...] = jnp.full((1, 8, _LANES), conf_i, f32)
    bbox_out_ref[...] = jnp.full((1, 8, _LANES), bbox_i, f32)


@jax.jit
def kernel(bbox_pred, conf_pred, anchors, target_boxes):
    B, N = conf_pred.shape
    pad = _NPAD - N
    ax = jnp.pad(anchors, ((0, pad), (0, 0))).T.reshape(4, _R, _LANES)
    bb = jnp.pad(bbox_pred, ((0, 0), (0, pad), (0, 0)))
    bb = bb.transpose(0, 2, 1).reshape(B, 4, _R, _LANES)
    cf = jnp.pad(conf_pred, ((0, 0), (0, pad)),
                 constant_values=0.5).reshape(B, 1, _R, _LANES)

    grid = (B,)
    conf_b, bbox_b = pl.pallas_call(
        _loss_kernel,
        grid=grid,
        in_specs=[
            pl.BlockSpec((1, _T, 4), lambda b: (b, 0, 0)),
            pl.BlockSpec((4, _R, _LANES), lambda b: (0, 0, 0)),
            pl.BlockSpec((1, 4, _R, _LANES), lambda b: (b, 0, 0, 0)),
            pl.BlockSpec((1, 1, _R, _LANES), lambda b: (b, 0, 0, 0)),
        ],
        out_specs=[
            pl.BlockSpec((1, 8, _LANES), lambda b: (b, 0, 0)),
            pl.BlockSpec((1, 8, _LANES), lambda b: (b, 0, 0)),
        ],
        out_shape=[
            jax.ShapeDtypeStruct((B, 8, _LANES), jnp.float32),
            jax.ShapeDtypeStruct((B, 8, _LANES), jnp.float32),
        ],
        scratch_shapes=[
            pltpu.VMEM((_R, _LANES), jnp.float32),
            pltpu.VMEM((_R, _LANES), jnp.int32),
            pltpu.VMEM((_R, _LANES), jnp.float32),
            pltpu.VMEM((_R, _LANES), jnp.int32),
        ],
    )(target_boxes, ax, bb.reshape(B, 4, _R, _LANES), cf)

    conf_loss = conf_b[:, 0, 0].mean()
    bbox_loss = bbox_b[:, 0, 0].mean()
    return (conf_loss + bbox_loss, conf_loss, bbox_loss)


# speedup vs baseline: 19.5521x; 1.0512x over previous
import functools

import jax
import jax.numpy as jnp
from jax.experimental import pallas as pl
from jax.experimental.pallas import tpu as pltpu

_N = 100000
_T = 20
_LANES = 128
_R = 784
_NPAD = _R * _LANES


def _loss_kernel(tb_ref, ax_ref, bbox_ref, conf_ref, conf_out_ref, bbox_out_ref,
                 miou_ref, bidx_ref, vm_ref, nbits_ref):
    f32 = jnp.float32
    row = jax.lax.broadcasted_iota(jnp.int32, (_R, _LANES), 0)
    col = jax.lax.broadcasted_iota(jnp.int32, (_R, _LANES), 1)
    lin = row * _LANES + col
    valid = lin < _N

    tb = tb_ref[0]
    ax1 = ax_ref[0]
    ay1 = ax_ref[1]
    ax2 = ax_ref[2]
    ay2 = ax_ref[3]
    a1 = (ax2 - ax1) * (ay2 - ay1)

    ibase = (jax.lax.broadcasted_iota(jnp.int32, (8, _LANES), 0) * _LANES
             + jax.lax.broadcasted_iota(jnp.int32, (8, _LANES), 1))
    ngroups = _R // 8
    top1_idx = []
    for t in range(_T):
        x1t = tb[t, 0]
        y1t = tb[t, 1]
        x2t = tb[t, 2]
        y2t = tb[t, 3]
        a2 = (x2t - x1t) * (y2t - y1t)
        ix1 = jnp.maximum(ax1, x1t)
        iy1 = jnp.maximum(ay1, y1t)
        ix2 = jnp.minimum(ax2, x2t)
        iy2 = jnp.minimum(ay2, y2t)
        inter = jnp.maximum(ix2 - ix1, 0.0) * jnp.maximum(iy2 - iy1, 0.0)
        union = a1 + a2 - inter
        iou_t = inter / (union + 1e-06)
        if t == 0:
            miou_ref[...] = iou_t
            bidx_ref[...] = jnp.zeros((_R, _LANES), jnp.int32)
        else:
            cur = miou_ref[...]
            upd = iou_t > cur
            miou_ref[...] = jnp.where(upd, iou_t, cur)
            bidx_ref[...] = jnp.where(upd, t, bidx_ref[...])
        vm_full = jnp.where(valid, iou_t, -1.0)
        vm_ref[...] = vm_full

        def sweep(g, carry):
            t1, t2, t3, i1p = carry
            x = vm_ref[pl.ds(g * 8, 8), :]
            upd1 = x > t1
            i1p = jnp.where(upd1, ibase + g * (8 * _LANES), i1p)
            b = jnp.minimum(t1, x)
            t1 = jnp.maximum(t1, x)
            c = jnp.minimum(t2, b)
            t2 = jnp.maximum(t2, b)
            t3 = jnp.maximum(t3, c)
            return t1, t2, t3, i1p

        neg2 = jnp.full((8, _LANES), -2.0)
        big = jnp.full((8, _LANES), _NPAD, jnp.int32)
        t1, t2, t3, i1p = jax.lax.fori_loop(
            0, ngroups, sweep, (neg2, neg2, neg2, big))

        m1 = jnp.max(t1)
        i1 = jnp.min(jnp.where(t1 == m1, i1p, _NPAD))
        top1_idx.append(i1)

        def next_distinct(val):
            m = jnp.float32(-2.0)
            for p in (t1, t2, t3):
                m = jnp.maximum(m, jnp.max(jnp.where(p < val, p, -2.0)))
            return m

        m2 = next_distinct(m1)
        m3 = next_distinct(m2)
        sel_t = (vm_full >= m3) & (vm_full > 0.3)
        if t == 0:
            possel = sel_t
        else:
            possel = possel | sel_t

    max_iou = miou_ref[...]
    pos = ((max_iou >= 0.5) & valid) | possel
    for i1 in top1_idx:
        pos = pos | (lin == i1)
    neg = (max_iou < 0.4) & (~pos) & valid

    conf = conf_ref[0, 0]
    pt = jnp.where(pos, conf, 1.0 - conf)
    af = jnp.where(pos, 0.25, 0.75)
    fl = -af * (1.0 - pt) ** 2 * jnp.log(jnp.clip(pt, 1e-06, 1.0 - 1e-06))

    num_pos = jnp.sum(pos.astype(jnp.int32))
    num_neg = jnp.sum(neg.astype(jnp.int32))
    sum_fl_pos = jnp.sum(jnp.where(pos, fl, 0.0))
    nbits_ref[...] = jnp.where(neg, jax.lax.bitcast_convert_type(fl, jnp.int32),
                               jnp.int32(-1))

    bidx = bidx_ref[...]
    mx1 = jnp.full((_R, _LANES), tb[0, 0])
    my1 = jnp.full((_R, _LANES), tb[0, 1])
    mx2 = jnp.full((_R, _LANES), tb[0, 2])
    my2 = jnp.full((_R, _LANES), tb[0, 3])
    for t in range(1, _T):
        m = bidx == t
        mx1 = jnp.where(m, tb[t, 0], mx1)
        my1 = jnp.where(m, tb[t, 1], my1)
        mx2 = jnp.where(m, tb[t, 2], mx2)
        my2 = jnp.where(m, tb[t, 3], my2)
    px1 = bbox_ref[0, 0]
    py1 = bbox_ref[0, 1]
    px2 = bbox_ref[0, 2]
    py2 = bbox_ref[0, 3]
    ix1 = jnp.maximum(px1, mx1)
    iy1 = jnp.maximum(py1, my1)
    ix2 = jnp.minimum(px2, mx2)
    iy2 = jnp.minimum(py2, my2)
    inter = jnp.maximum(ix2 - ix1, 0.0) * jnp.maximum(iy2 - iy1, 0.0)
    pa = (px2 - px1) * (py2 - py1)
    ma = (mx2 - mx1) * (my2 - my1)
    union = pa + ma - inter
    iou = inter / (union + 1e-06)
    ex1 = jnp.minimum(px1, mx1)
    ey1 = jnp.minimum(py1, my1)
    ex2 = jnp.maximum(px2, mx2)
    ey2 = jnp.maximum(py2, my2)
    enc = (ex2 - ex1) * (ey2 - ey1)
    giou = iou - (enc - union) / (enc + 1e-06)
    l1 = (jnp.abs(px1 - mx1) + jnp.abs(py1 - my1) + jnp.abs(px2 - mx2)
          + jnp.abs(py2 - my2)) * 0.25
    per_anchor = (1.0 - giou) + 0.5 * l1
    sum_bbox_pos = jnp.sum(jnp.where(pos, per_anchor, 0.0))

    safe_np = jnp.maximum(num_pos, 1)
    neg_ratio = jnp.minimum(3, _N // safe_np)
    max_neg = neg_ratio * num_pos
    k1 = jnp.minimum(max_neg, num_neg)
    k2 = jnp.minimum(100, num_neg)

    def bs_body(_, carry):
        lo1, hi1, lo2, hi2 = carry
        m1 = lo1 + (hi1 - lo1) // 2
        m2 = lo2 + (hi2 - lo2) // 2
        nb = nbits_ref[...]
        c1 = jnp.sum((nb >= m1).astype(jnp.int32))
        c2 = jnp.sum((nb >= m2).astype(jnp.int32))
        t1 = c1 >= k1
        t2 = c2 >= k2
        return (jnp.where(t1, m1, lo1), jnp.where(t1, hi1, m1),
                jnp.where(t2, m2, lo2), jnp.where(t2, hi2, m2))

    z = jnp.int32(0)
    top = jnp.int32(0x7F800001)
    lo1, _, lo2, _ = jax.lax.fori_loop(0, 31, bs_body, (z, top, z, top))

    nb = nbits_ref[...]
    nv = jnp.where(nb >= 0, jax.lax.bitcast_convert_type(nb, f32), 0.0)
    v1 = jax.lax.bitcast_convert_type(lo1, f32)
    v2 = jax.lax.bitcast_convert_type(lo2, f32)
    g1 = nb > lo1
    g2 = nb > lo2
    cnt_g1 = jnp.sum(g1.astype(jnp.int32))
    cnt_g2 = jnp.sum(g2.astype(jnp.int32))
    sum_g1 = jnp.sum(jnp.where(g1, nv, 0.0))
    sum_g2 = jnp.sum(jnp.where(g2, nv, 0.0))
    sum_top1 = jnp.where(k1 > 0, sum_g1 + (k1 - cnt_g1).astype(f32) * v1, 0.0)
    sum_top2 = jnp.where(k2 > 0, sum_g2 + (k2 - cnt_g2).astype(f32) * v2, 0.0)

    conf_pos = (sum_fl_pos + sum_top1) / (num_pos + k1).astype(f32)
    conf_neg = sum_top2 / k2.astype(f32)
    conf_i = jnp.where(num_pos > 0, conf_pos, conf_neg)
    bbox_i = jnp.where(num_pos > 0,
                       sum_bbox_pos / safe_np.astype(f32), 0.0)
    conf_out_ref[...] = jnp.full((1, 8, _LANES), conf_i, f32)
    bbox_out_ref[...] = jnp.full((1, 8, _LANES), bbox_i, f32)


@jax.jit
def kernel(bbox_pred, conf_pred, anchors, target_boxes):
    B, N = conf_pred.shape
    pad = _NPAD - N
    ax = jnp.pad(anchors, ((0, pad), (0, 0))).T.reshape(4, _R, _LANES)
    bb = jnp.pad(bbox_pred, ((0, 0), (0, pad), (0, 0)))
    bb = bb.transpose(0, 2, 1).reshape(B, 4, _R, _LANES)
    cf = jnp.pad(conf_pred, ((0, 0), (0, pad)),
                 constant_values=0.5).reshape(B, 1, _R, _LANES)

    grid = (B,)
    conf_b, bbox_b = pl.pallas_call(
        _loss_kernel,
        grid=grid,
        in_specs=[
            pl.BlockSpec((1, _T, 4), lambda b: (b, 0, 0)),
            pl.BlockSpec((4, _R, _LANES), lambda b: (0, 0, 0)),
            pl.BlockSpec((1, 4, _R, _LANES), lambda b: (b, 0, 0, 0)),
            pl.BlockSpec((1, 1, _R, _LANES), lambda b: (b, 0, 0, 0)),
        ],
        out_specs=[
            pl.BlockSpec((1, 8, _LANES), lambda b: (b, 0, 0)),
            pl.BlockSpec((1, 8, _LANES), lambda b: (b, 0, 0)),
        ],
        out_shape=[
            jax.ShapeDtypeStruct((B, 8, _LANES), jnp.float32),
            jax.ShapeDtypeStruct((B, 8, _LANES), jnp.float32),
        ],
        scratch_shapes=[
            pltpu.VMEM((_R, _LANES), jnp.float32),
            pltpu.VMEM((_R, _LANES), jnp.int32),
            pltpu.VMEM((_R, _LANES), jnp.float32),
            pltpu.VMEM((_R, _LANES), jnp.int32),
        ],
    )(target_boxes, ax, bb.reshape(B, 4, _R, _LANES), cf)

    conf_loss = conf_b[:, 0, 0].mean()
    bbox_loss = bbox_b[:, 0, 0].mean()
    return (conf_loss + bbox_loss, conf_loss, bbox_loss)
